# baseline (speedup 1.0000x reference)
import jax
import jax.numpy as jnp
from jax import lax
from jax.experimental import pallas as pl
from jax.experimental.pallas import tpu as pltpu

N = 4096
D_IN = 256
D_OUT = 64
NHEADS = 4
ALPHA = 0.2
BR = 1024
LOG2E = 1.4426950408889634
HSLOT = 128


def _gat_kernel(x_ref, adj_ref,
                w0_ref, a0_ref, w1_ref, a1_ref, w2_ref, a2_ref, w3_ref, a3_ref,
                out_ref,
                whx_ref, e1_ref, e2t_ref):
    i = pl.program_id(0)
    w_refs = (w0_ref, w1_ref, w2_ref, w3_ref)
    a_refs = (a0_ref, a1_ref, a2_ref, a3_ref)

    @pl.when(i == 0)
    def _init():
        x = x_ref[...]
        whx_ref[...] = jnp.zeros((N, NHEADS * HSLOT), jnp.bfloat16)
        ones_col = jnp.ones((N, 1), jnp.bfloat16)
        for h in range(NHEADS):
            wh = jnp.dot(x, w_refs[h][...], preferred_element_type=jnp.float32)
            e1_ref[:, h:h + 1] = LOG2E * jnp.dot(
                wh, a_refs[h][0:D_OUT, :], preferred_element_type=jnp.float32)
            e2t_ref[h:h + 1, :] = LOG2E * lax.dot_general(
                a_refs[h][D_OUT:2 * D_OUT, :], wh,
                (((0,), (1,)), ((), ())),
                preferred_element_type=jnp.float32)
            whx_ref[:, h * HSLOT:h * HSLOT + D_OUT] = wh.astype(jnp.bfloat16)
            whx_ref[:, h * HSLOT + D_OUT:h * HSLOT + D_OUT + 1] = ones_col

    madj = adj_ref[...].astype(jnp.bfloat16)
    e1_blk = e1_ref[pl.ds(i * BR, BR), :]
    for h in range(NHEADS):
        y = e1_blk[:, h:h + 1] + e2t_ref[h:h + 1, :]
        u = jnp.maximum(y, ALPHA * y)
        p = jnp.exp2(u).astype(jnp.bfloat16) * madj
        o = jnp.dot(p, whx_ref[:, h * HSLOT:(h + 1) * HSLOT],
                    preferred_element_type=jnp.float32)
        out_ref[:, h * D_OUT:(h + 1) * D_OUT] = (
            o[:, 0:D_OUT] / o[:, D_OUT:D_OUT + 1])


def kernel(x, adj, W0, a0, W1, a1, W2, a2, W3, a3):
    grid = (N // BR,)
    resident = lambda shape: pl.BlockSpec(shape, lambda i: (0, 0))
    out = pl.pallas_call(
        _gat_kernel,
        grid=grid,
        in_specs=[
            resident((N, D_IN)),
            pl.BlockSpec((BR, N), lambda i: (i, 0)),
            resident((D_IN, D_OUT)), resident((2 * D_OUT, 1)),
            resident((D_IN, D_OUT)), resident((2 * D_OUT, 1)),
            resident((D_IN, D_OUT)), resident((2 * D_OUT, 1)),
            resident((D_IN, D_OUT)), resident((2 * D_OUT, 1)),
        ],
        out_specs=pl.BlockSpec((BR, NHEADS * D_OUT), lambda i: (i, 0)),
        out_shape=jax.ShapeDtypeStruct((N, NHEADS * D_OUT), jnp.float32),
        scratch_shapes=[
            pltpu.VMEM((N, NHEADS * HSLOT), jnp.bfloat16),
            pltpu.VMEM((N, NHEADS), jnp.float32),
            pltpu.VMEM((NHEADS, N), jnp.float32),
        ],
        compiler_params=pltpu.CompilerParams(
            dimension_semantics=("arbitrary",),
        ),
    )(x, adj, W0, a0, W1, a1, W2, a2, W3, a3)
    return out


# speedup vs baseline: 1.0202x; 1.0202x over previous
import jax
import jax.numpy as jnp
from jax import lax
from jax.experimental import pallas as pl
from jax.experimental.pallas import tpu as pltpu

N = 4096
D_IN = 256
D_OUT = 64
NHEADS = 4
ALPHA = 0.2
BR = 512
LOG2E = 1.4426950408889634
HSLOT = 128


def _gat_kernel(x_ref, adj_ref,
                w0_ref, a0_ref, w1_ref, a1_ref, w2_ref, a2_ref, w3_ref, a3_ref,
                out_ref,
                whx_ref, e1_ref, e2t_ref):
    i = pl.program_id(0)
    w_refs = (w0_ref, w1_ref, w2_ref, w3_ref)
    a_refs = (a0_ref, a1_ref, a2_ref, a3_ref)

    @pl.when(i == 0)
    def _init():
        xb = x_ref[...].astype(jnp.bfloat16)
        ones_col = jnp.ones((N, 1), jnp.bfloat16)
        for h in range(NHEADS):
            wh = jnp.dot(xb, w_refs[h][...].astype(jnp.bfloat16),
                         preferred_element_type=jnp.float32)
            e1_ref[:, h:h + 1] = LOG2E * jnp.dot(
                wh, a_refs[h][0:D_OUT, :], preferred_element_type=jnp.float32)
            e2t_ref[h:h + 1, :] = LOG2E * lax.dot_general(
                a_refs[h][D_OUT:2 * D_OUT, :], wh,
                (((0,), (1,)), ((), ())),
                preferred_element_type=jnp.float32)
            whx_ref[:, h * HSLOT:h * HSLOT + D_OUT] = wh.astype(jnp.bfloat16)
            whx_ref[:, h * HSLOT + D_OUT:h * HSLOT + D_OUT + 1] = ones_col

    madj = adj_ref[...].astype(jnp.bfloat16)
    e1_blk = e1_ref[pl.ds(i * BR, BR), :]
    for h in range(NHEADS):
        y = e1_blk[:, h:h + 1] + e2t_ref[h:h + 1, :]
        u = jnp.maximum(y, ALPHA * y)
        p = jnp.exp2(u).astype(jnp.bfloat16) * madj
        o = jnp.dot(p, whx_ref[:, h * HSLOT:(h + 1) * HSLOT],
                    preferred_element_type=jnp.float32)
        out_ref[:, h * D_OUT:(h + 1) * D_OUT] = (
            o[:, 0:D_OUT] / o[:, D_OUT:D_OUT + 1])


def kernel(x, adj, W0, a0, W1, a1, W2, a2, W3, a3):
    grid = (N // BR,)
    resident = lambda shape: pl.BlockSpec(shape, lambda i: (0, 0))
    out = pl.pallas_call(
        _gat_kernel,
        grid=grid,
        in_specs=[
            resident((N, D_IN)),
            pl.BlockSpec((BR, N), lambda i: (i, 0)),
            resident((D_IN, D_OUT)), resident((2 * D_OUT, 1)),
            resident((D_IN, D_OUT)), resident((2 * D_OUT, 1)),
            resident((D_IN, D_OUT)), resident((2 * D_OUT, 1)),
            resident((D_IN, D_OUT)), resident((2 * D_OUT, 1)),
        ],
        out_specs=pl.BlockSpec((BR, NHEADS * D_OUT), lambda i: (i, 0)),
        out_shape=jax.ShapeDtypeStruct((N, NHEADS * D_OUT), jnp.float32),
        scratch_shapes=[
            pltpu.VMEM((N, NHEADS * HSLOT), jnp.bfloat16),
            pltpu.VMEM((N, NHEADS), jnp.float32),
            pltpu.VMEM((NHEADS, N), jnp.float32),
        ],
        compiler_params=pltpu.CompilerParams(
            dimension_semantics=("arbitrary",),
        ),
    )(x, adj, W0, a0, W1, a1, W2, a2, W3, a3)
    return out


# speedup vs baseline: 1.1900x; 1.1664x over previous
import jax
import jax.numpy as jnp
from jax import lax
from jax.experimental import pallas as pl
from jax.experimental.pallas import tpu as pltpu

N = 4096
D_IN = 256
D_OUT = 64
NHEADS = 4
ALPHA = 0.2
BR = 512
LOG2E = 1.4426950408889634
HSLOT = 128


def _gat_kernel(x_ref, adj_ref,
                w0_ref, a0_ref, w1_ref, a1_ref, w2_ref, a2_ref, w3_ref, a3_ref,
                out_ref,
                whx_ref, e1_ref, e2t_ref):
    i = pl.program_id(0)
    w_refs = (w0_ref, w1_ref, w2_ref, w3_ref)
    a_refs = (a0_ref, a1_ref, a2_ref, a3_ref)

    @pl.when(i == 0)
    def _init():
        xb = x_ref[...].astype(jnp.bfloat16)
        ones_col = jnp.ones((N, 1), jnp.bfloat16)
        for h in range(NHEADS):
            wh = jnp.dot(xb, w_refs[h][...].astype(jnp.bfloat16),
                         preferred_element_type=jnp.float32)
            e1_ref[:, h:h + 1] = (LOG2E * jnp.dot(
                wh, a_refs[h][0:D_OUT, :],
                preferred_element_type=jnp.float32)).astype(jnp.bfloat16)
            e2t_ref[h:h + 1, :] = (LOG2E * lax.dot_general(
                a_refs[h][D_OUT:2 * D_OUT, :], wh,
                (((0,), (1,)), ((), ())),
                preferred_element_type=jnp.float32)).astype(jnp.bfloat16)
            whx_ref[:, h * HSLOT:h * HSLOT + D_OUT] = wh.astype(jnp.bfloat16)
            whx_ref[:, h * HSLOT + D_OUT:h * HSLOT + D_OUT + 1] = ones_col

    madj = adj_ref[...].astype(jnp.bfloat16)
    e1_blk = e1_ref[pl.ds(i * BR, BR), :]
    for h in range(NHEADS):
        y = e1_blk[:, h:h + 1] + e2t_ref[h:h + 1, :]
        u = jnp.maximum(y, jnp.bfloat16(ALPHA) * y)
        p = jnp.exp2(u) * madj
        o = jnp.dot(p, whx_ref[:, h * HSLOT:(h + 1) * HSLOT],
                    preferred_element_type=jnp.float32)
        out_ref[:, h * D_OUT:(h + 1) * D_OUT] = (
            o[:, 0:D_OUT] / o[:, D_OUT:D_OUT + 1])


def kernel(x, adj, W0, a0, W1, a1, W2, a2, W3, a3):
    grid = (N // BR,)
    resident = lambda shape: pl.BlockSpec(shape, lambda i: (0, 0))
    out = pl.pallas_call(
        _gat_kernel,
        grid=grid,
        in_specs=[
            resident((N, D_IN)),
            pl.BlockSpec((BR, N), lambda i: (i, 0)),
            resident((D_IN, D_OUT)), resident((2 * D_OUT, 1)),
            resident((D_IN, D_OUT)), resident((2 * D_OUT, 1)),
            resident((D_IN, D_OUT)), resident((2 * D_OUT, 1)),
            resident((D_IN, D_OUT)), resident((2 * D_OUT, 1)),
        ],
        out_specs=pl.BlockSpec((BR, NHEADS * D_OUT), lambda i: (i, 0)),
        out_shape=jax.ShapeDtypeStruct((N, NHEADS * D_OUT), jnp.float32),
        scratch_shapes=[
            pltpu.VMEM((N, NHEADS * HSLOT), jnp.bfloat16),
            pltpu.VMEM((N, NHEADS), jnp.bfloat16),
            pltpu.VMEM((NHEADS, N), jnp.bfloat16),
        ],
        compiler_params=pltpu.CompilerParams(
            dimension_semantics=("arbitrary",),
        ),
    )(x, adj, W0, a0, W1, a1, W2, a2, W3, a3)
    return out
